# baseline (speedup 1.0000x reference)
import functools

import jax
import jax.numpy as jnp
import numpy as np
from jax import lax
from jax.experimental import pallas as pl
from jax.experimental.pallas import tpu as pltpu
from jax.experimental.pallas import tpu_sc as plsc

B, C, T, H, W = 8, 256, 16, 28, 28
D = H * W
R = B * C * T
NW = 32
RPW = R // NW
CHUNK = 64
NCH = RPW // CHUNK
LANES = 16
NVREG = D // LANES

_idx = np.arange(D)
_i, _j = _idx // W, _idx % W
_valid = ((_j != 0) & (_i != H - 1) & (_j != W - 1)).astype(np.float32)
_NVALID = int(_valid.sum())

_WPLAN = []
_WCLASSES = {}
for _k in range(NVREG):
    seg = _valid[_k * LANES:(_k + 1) * LANES]
    if seg.min() == 1.0:
        _WPLAN.append((_k, None))
    elif seg.max() == 0.0:
        continue
    else:
        _cls = _k if _k * LANES + LANES > 756 else _k % 7
        _WCLASSES.setdefault(_cls, _k)
        _WPLAN.append((_k, _cls))

def _lane_shuffle(vec, idx):
    return lax.gather(
        vec,
        idx[:, None],
        lax.GatherDimensionNumbers(
            offset_dims=(),
            collapsed_slice_dims=(0,),
            start_index_map=(0,),
        ),
        slice_sizes=(1,),
        mode=lax.GatherScatterMode.PROMISE_IN_BOUNDS,
    )


@functools.cache
def _build_sc_pool():
    mesh = plsc.VectorSubcoreMesh(core_axis_name="c", subcore_axis_name="s")
    return pl.kernel(
        _sc_pool_body,
        mesh=mesh,
        out_type=jax.ShapeDtypeStruct((R,), jnp.float32),
        scratch_types=[
            pltpu.VMEM((2, CHUNK, D), jnp.float32),
            pltpu.VMEM((RPW,), jnp.float32),
            pltpu.SemaphoreType.DMA,
            pltpu.SemaphoreType.DMA,
        ],
    )


def _sc_pool_body(x_hbm, out_hbm, buf, outv, sem0, sem1):
    wid = lax.axis_index("c") * 16 + lax.axis_index("s")
    base = wid * RPW
    sems = (sem0, sem1)
    inv = jnp.float32(1.0 / _NVALID)

    iota = lax.iota(jnp.int32, LANES)
    wvecs = {}
    for cls, krep in _WCLASSES.items():
        col = iota + krep * LANES
        jmod = col % W
        m = (jmod != 0) & (jmod != W - 1) & (col < (H - 1) * W)
        wvecs[cls] = jnp.where(m, jnp.float32(1.0), jnp.float32(0.0))

    bfly = [iota ^ s for s in (8, 4, 2, 1)]
    zero16 = jnp.broadcast_to(jnp.float32(0.0), (LANES,))

    def copy(c, b):
        return pltpu.make_async_copy(
            x_hbm.at[pl.ds(base + c * CHUNK, CHUNK)], buf.at[b], sems[b]
        )

    copy(0, 0).start()
    copy(1, 1).start()

    for c in range(NCH):
        b = c & 1

        copy(c, b).wait()

        def group_body(g, carry, b=b, c=c):
            def row_body(p, ovec, g=g, b=b):
                r = g * LANES + p
                parts = []
                for q in range(4):
                    acc = None
                    for k, cls in _WPLAN[q::4]:
                        v = buf[b, r, pl.ds(k * LANES, LANES)]
                        term = v if cls is None else v * wvecs[cls]
                        acc = term if acc is None else acc + term
                    parts.append(acc)
                t = ((parts[0] + parts[1]) + (parts[2] + parts[3])) * inv
                for pidx in bfly:
                    t = t + _lane_shuffle(t, pidx)
                return jnp.where(iota == p, t, ovec)

            ovec = lax.fori_loop(0, LANES, row_body, zero16)
            outv[pl.ds(c * CHUNK + g * LANES, LANES)] = ovec
            return carry

        lax.fori_loop(0, CHUNK // LANES, group_body, 0)

        if c + 2 < NCH:
            copy(c + 2, b).start()

    pltpu.sync_copy(outv, out_hbm.at[pl.ds(base, RPW)])


@jax.jit
def kernel(x):
    out = _build_sc_pool()(x.reshape(R, D))
    return out.reshape(B, C, T)


# speedup vs baseline: 1.2852x; 1.2852x over previous
import functools

import jax
import jax.numpy as jnp
import numpy as np
from jax import lax
from jax.experimental import pallas as pl
from jax.experimental.pallas import tpu as pltpu
from jax.experimental.pallas import tpu_sc as plsc

B, C, T, H, W = 8, 256, 16, 28, 28
R = B * C * T
BLK = 128
_NVALID = (H - 1) * (W - 2)

def _tc_body(x_ref, o_ref):
    blk = x_ref[...]
    jj = lax.broadcasted_iota(jnp.int32, (BLK, H, W), 2)
    y = jnp.sum(jnp.where((jj >= 1) & (jj < W - 1), blk, 0.0), axis=2)
    ii = lax.broadcasted_iota(jnp.int32, (BLK, H), 1)
    z = jnp.sum(jnp.where(ii < H - 1, y, 0.0), axis=1)
    o_ref[...] = z * jnp.float32(1.0 / _NVALID)


@functools.cache
def _build_tc_pool():
    return pl.pallas_call(
        _tc_body,
        grid=(R // BLK,),
        in_specs=[pl.BlockSpec((BLK, H, W), lambda i: (i, 0, 0))],
        out_specs=pl.BlockSpec((BLK,), lambda i: (i,)),
        out_shape=jax.ShapeDtypeStruct((R,), jnp.float32),
    )


@jax.jit
def kernel(x):
    out = _build_tc_pool()(x.reshape(R, H, W))
    return out.reshape(B, C, T)


# speedup vs baseline: 1.5726x; 1.2236x over previous
import functools

import jax
import jax.numpy as jnp
import numpy as np
from jax import lax
from jax.experimental import pallas as pl
from jax.experimental.pallas import tpu as pltpu
from jax.experimental.pallas import tpu_sc as plsc

B, C, T, H, W = 8, 256, 16, 28, 28
R = B * C * T
NW = 32
RPW = R // NW
CHUNK = 8
NCH = RPW // CHUNK
LANES = 16
_NVALID = (H - 1) * (W - 2)


def _lane_shuffle(vec, idx):
    return lax.gather(
        vec,
        idx[:, None],
        lax.GatherDimensionNumbers(
            offset_dims=(),
            collapsed_slice_dims=(0,),
            start_index_map=(0,),
        ),
        slice_sizes=(1,),
        mode=lax.GatherScatterMode.PROMISE_IN_BOUNDS,
    )


def _sc_pool_body(x_hbm, out_hbm, buf, outv, sem0, sem1):
    wid = lax.axis_index("c") * 16 + lax.axis_index("s")
    base = wid * RPW
    sems = (sem0, sem1)
    inv = jnp.float32(1.0 / _NVALID)

    iota = lax.iota(jnp.int32, LANES)
    zero16 = jnp.broadcast_to(jnp.float32(0.0), (LANES,))
    mask_a = iota != 0
    mask_c = (iota >= 4) & (iota != LANES - 1)
    bfly = [iota ^ s for s in (8, 4, 2, 1)]

    def copy(c, b):
        return pltpu.make_async_copy(
            x_hbm.at[pl.ds(base + c * CHUNK, CHUNK)], buf.at[b], sems[b]
        )

    def compute_chunk(b, ovec0, poff):
        def row_body(p, ovec):
            accs = [None] * 4
            for i in range(H - 1):
                va = jnp.where(mask_a, buf[b, p, i, pl.ds(0, LANES)], 0.0)
                vc = jnp.where(mask_c, buf[b, p, i, pl.ds(W - LANES, LANES)], 0.0)
                qa, qb = (2 * i) & 3, (2 * i + 1) & 3
                accs[qa] = va if accs[qa] is None else accs[qa] + va
                accs[qb] = vc if accs[qb] is None else accs[qb] + vc
            t = ((accs[0] + accs[1]) + (accs[2] + accs[3])) * inv
            for pidx in bfly:
                t = t + _lane_shuffle(t, pidx)
            return jnp.where(iota == p + poff, t, ovec)

        return lax.fori_loop(0, CHUNK, row_body, ovec0)

    copy(jnp.int32(0), 0).start()
    copy(jnp.int32(1), 1).start()

    def pair_body(e, carry):
        c0 = e * 2
        copy(c0, 0).wait()
        ovec = compute_chunk(0, zero16, 0)

        @pl.when(c0 + 2 < NCH)
        def _():
            copy(c0 + 2, 0).start()

        copy(c0 + 1, 1).wait()
        ovec = compute_chunk(1, ovec, CHUNK)

        @pl.when(c0 + 3 < NCH)
        def _():
            copy(c0 + 3, 1).start()

        outv[pl.ds(e * LANES, LANES)] = ovec
        return carry

    lax.fori_loop(0, NCH // 2, pair_body, 0)

    pltpu.sync_copy(outv, out_hbm.at[pl.ds(base, RPW)])


@functools.cache
def _build_sc_pool():
    mesh = plsc.VectorSubcoreMesh(core_axis_name="c", subcore_axis_name="s")
    return pl.kernel(
        _sc_pool_body,
        mesh=mesh,
        out_type=jax.ShapeDtypeStruct((R,), jnp.float32),
        scratch_types=[
            pltpu.VMEM((2, CHUNK, H, W), jnp.float32),
            pltpu.VMEM((RPW,), jnp.float32),
            pltpu.SemaphoreType.DMA,
            pltpu.SemaphoreType.DMA,
        ],
        compiler_params=pltpu.CompilerParams(use_tc_tiling_on_sc=True),
    )


@jax.jit
def kernel(x):
    out = _build_sc_pool()(x.reshape(R, H, W))
    return out.reshape(B, C, T)
